# baseline (speedup 1.0000x reference)
import jax
import jax.numpy as jnp
from jax.experimental import pallas as pl
from jax.experimental.pallas import tpu as pltpu


_N_STRIPES = 16


def _copy_body(src_ref, dst_ref, sems):
    rows = src_ref.shape[0]
    stripe = rows // _N_STRIPES
    copies = [
        pltpu.make_async_copy(
            src_ref.at[pl.ds(i * stripe, stripe)],
            dst_ref.at[0, pl.ds(i * stripe, stripe)],
            sems.at[i],
        )
        for i in range(_N_STRIPES)
    ]
    for c in copies:
        c.start()
    for c in copies:
        c.wait()


def kernel(x, embedding_weight):
    seq = x.shape[1]
    dim = embedding_weight.shape[1]
    return pl.pallas_call(
        _copy_body,
        out_shape=jax.ShapeDtypeStruct((1, seq, dim), embedding_weight.dtype),
        in_specs=[pl.BlockSpec(memory_space=pltpu.MemorySpace.HBM)],
        out_specs=pl.BlockSpec(memory_space=pltpu.MemorySpace.HBM),
        scratch_shapes=[pltpu.SemaphoreType.DMA((_N_STRIPES,))],
    )(embedding_weight[:seq])


# speedup vs baseline: 41.3390x; 41.3390x over previous
import jax
import jax.numpy as jnp
from jax.experimental import pallas as pl
from jax.experimental.pallas import tpu as pltpu


_BLOCK_ROWS = 512


def _copy_body(src_ref, dst_ref):
    dst_ref[...] = src_ref[...][None]


def kernel(x, embedding_weight):
    seq = x.shape[1]
    dim = embedding_weight.shape[1]
    grid = (seq // _BLOCK_ROWS,)
    return pl.pallas_call(
        _copy_body,
        out_shape=jax.ShapeDtypeStruct((1, seq, dim), embedding_weight.dtype),
        grid=grid,
        in_specs=[pl.BlockSpec((_BLOCK_ROWS, dim), lambda i: (i, 0))],
        out_specs=pl.BlockSpec((1, _BLOCK_ROWS, dim), lambda i: (0, i, 0)),
    )(embedding_weight[:seq])


# speedup vs baseline: 44.9480x; 1.0873x over previous
import jax
import jax.numpy as jnp
from jax.experimental import pallas as pl
from jax.experimental.pallas import tpu as pltpu


_BLOCK_ROWS = 1024


def _copy_body(src_ref, dst_ref):
    dst_ref[...] = src_ref[...][None]


def kernel(x, embedding_weight):
    seq = x.shape[1]
    dim = embedding_weight.shape[1]
    grid = (seq // _BLOCK_ROWS,)
    return pl.pallas_call(
        _copy_body,
        out_shape=jax.ShapeDtypeStruct((1, seq, dim), embedding_weight.dtype),
        grid=grid,
        in_specs=[pl.BlockSpec((_BLOCK_ROWS, dim), lambda i: (i, 0))],
        out_specs=pl.BlockSpec((1, _BLOCK_ROWS, dim), lambda i: (0, i, 0)),
    )(embedding_weight[:seq])


# speedup vs baseline: 48.6428x; 1.0822x over previous
import jax
import jax.numpy as jnp
from jax.experimental import pallas as pl
from jax.experimental.pallas import tpu as pltpu


_BLOCK_ROWS = 2048


def _copy_body(src_ref, dst_ref):
    dst_ref[...] = src_ref[...][None]


def kernel(x, embedding_weight):
    seq = x.shape[1]
    dim = embedding_weight.shape[1]
    grid = (seq // _BLOCK_ROWS,)
    return pl.pallas_call(
        _copy_body,
        out_shape=jax.ShapeDtypeStruct((1, seq, dim), embedding_weight.dtype),
        grid=grid,
        in_specs=[pl.BlockSpec((_BLOCK_ROWS, dim), lambda i: (i, 0))],
        out_specs=pl.BlockSpec((1, _BLOCK_ROWS, dim), lambda i: (0, i, 0)),
    )(embedding_weight[:seq])
